# baseline (speedup 1.0000x reference)
import functools

import jax
import jax.numpy as jnp
from jax import lax
from jax.experimental import pallas as pl
from jax.experimental.pallas import tpu as pltpu
from jax.experimental.pallas import tpu_sc as plsc

D = 64
SCALE = 8.0


@functools.lru_cache(maxsize=None)
def _make_sc_kernel(B: int, chunk: int):
    info = plsc.get_sparse_core_info()
    nw = info.num_cores * info.num_subcores
    b_per_w = B // nw
    n_chunks = b_per_w // chunk
    mesh = plsc.VectorSubcoreMesh(core_axis_name="c", subcore_axis_name="s")

    @functools.partial(
        pl.kernel,
        mesh=mesh,
        out_type=jax.ShapeDtypeStruct((B, D), jnp.float32),
        compiler_params=pltpu.CompilerParams(use_tc_tiling_on_sc=False),
        scratch_types=[
            pltpu.VMEM((chunk,), jnp.int32),
            pltpu.VMEM((chunk, D), jnp.float32),
            pltpu.SemaphoreType.DMA,
        ],
    )
    def k(idx_hbm, table_hbm, out_hbm, idx_v, rows_v, sem):
        wid = lax.axis_index("s") * info.num_cores + lax.axis_index("c")
        base = wid * b_per_w

        def body(g, carry):
            off = base + g * chunk
            pltpu.sync_copy(idx_hbm.at[pl.ds(off, chunk)], idx_v)
            pltpu.async_copy(table_hbm.at[idx_v], rows_v, sem).wait()

            def scale_row(i, c):
                for j in range(D // 16):
                    sl = pl.ds(j * 16, 16)
                    rows_v[i, sl] = rows_v[i, sl] * SCALE
                return c

            lax.fori_loop(0, chunk, scale_row, 0)
            pltpu.sync_copy(rows_v, out_hbm.at[pl.ds(off, chunk)])
            return carry

        lax.fori_loop(0, n_chunks, body, 0)

    return k


def kernel(xb, table):
    r, c = xb.shape
    B = r * c
    idx = xb.reshape(B).astype(jnp.int32)
    out = _make_sc_kernel(B, 128)(idx, table)
    return out.reshape(r, c, D)


# speedup vs baseline: 1.2663x; 1.2663x over previous
import functools

import jax
import jax.numpy as jnp
from jax import lax
from jax.experimental import pallas as pl
from jax.experimental.pallas import tpu as pltpu
from jax.experimental.pallas import tpu_sc as plsc

D = 64
SCALE = 8.0
CHUNK = 128
NBUF = 4


@functools.lru_cache(maxsize=None)
def _make_sc_kernel(B: int):
    info = plsc.get_sparse_core_info()
    nw = info.num_cores * info.num_subcores
    b_per_w = B // nw
    n_chunks = b_per_w // CHUNK
    n_quads = n_chunks // NBUF
    mesh = plsc.VectorSubcoreMesh(core_axis_name="c", subcore_axis_name="s")

    @functools.partial(
        pl.kernel,
        mesh=mesh,
        out_type=jax.ShapeDtypeStruct((B, D), jnp.float32),
        compiler_params=pltpu.CompilerParams(use_tc_tiling_on_sc=False),
        scratch_types=[
            pltpu.VMEM((b_per_w,), jnp.int32),
            [pltpu.VMEM((CHUNK, D), jnp.float32) for _ in range(NBUF)],
            [pltpu.SemaphoreType.DMA for _ in range(NBUF)],
            [pltpu.SemaphoreType.DMA for _ in range(NBUF)],
        ],
    )
    def k(idx_hbm, table_hbm, out_hbm, idx_all, rows, gsems, osems):
        wid = lax.axis_index("s") * info.num_cores + lax.axis_index("c")
        base = wid * b_per_w
        pltpu.sync_copy(idx_hbm.at[pl.ds(base, b_per_w)], idx_all)

        def gather_copy(g, b):
            idx_sl = idx_all.at[pl.ds(g * CHUNK, CHUNK)]
            return pltpu.make_async_copy(table_hbm.at[idx_sl], rows[b], gsems[b])

        def out_copy(g, b):
            dst = out_hbm.at[pl.ds(base + g * CHUNK, CHUNK)]
            return pltpu.make_async_copy(rows[b], dst, osems[b])

        def scale_buf(b):
            buf = rows[b]

            @plsc.parallel_loop(0, CHUNK, unroll=4)
            def _(i):
                for j in range(D // 16):
                    sl = pl.ds(j * 16, 16)
                    buf[i, sl] = buf[i, sl] * SCALE

        for b in range(NBUF):
            gather_copy(b, b).start()

        def body(q, carry):
            g0 = q * NBUF
            for b in range(NBUF):
                gather_copy(g0 + b, b).wait()
                scale_buf(b)
                out_copy(g0 + b, b).start()
            for b in range(NBUF):
                out_copy(g0 + b, b).wait()
                gather_copy(g0 + NBUF + b, b).start()
            return carry

        lax.fori_loop(0, n_quads - 1, body, 0)

        g0 = (n_quads - 1) * NBUF
        for b in range(NBUF):
            gather_copy(g0 + b, b).wait()
            scale_buf(b)
            out_copy(g0 + b, b).start()
        for b in range(NBUF):
            out_copy(g0 + b, b).wait()

    return k


def kernel(xb, table):
    r, c = xb.shape
    B = r * c
    idx = xb.reshape(B).astype(jnp.int32)
    out = _make_sc_kernel(B)(idx, table)
    return out.reshape(r, c, D)


# speedup vs baseline: 1.2703x; 1.0032x over previous
import functools

import jax
import jax.numpy as jnp
from jax import lax
from jax.experimental import pallas as pl
from jax.experimental.pallas import tpu as pltpu
from jax.experimental.pallas import tpu_sc as plsc

D = 64
SCALE = 8.0
NBUF = 4
SPLIT = 128


@functools.lru_cache(maxsize=None)
def _make_sc_kernel(nseq_total: int, seq: int):
    info = plsc.get_sparse_core_info()
    nw = info.num_cores * info.num_subcores
    seq_per_w = nseq_total // nw
    n_quads = seq_per_w // NBUF
    rest = seq - SPLIT
    mesh = plsc.VectorSubcoreMesh(core_axis_name="c", subcore_axis_name="s")

    @functools.partial(
        pl.kernel,
        mesh=mesh,
        out_type=jax.ShapeDtypeStruct((nseq_total, seq, D), jnp.float32),
        compiler_params=pltpu.CompilerParams(use_tc_tiling_on_sc=False),
        scratch_types=[
            pltpu.VMEM((seq_per_w, seq), jnp.int32),
            [pltpu.VMEM((seq, D), jnp.float32) for _ in range(NBUF)],
            [pltpu.SemaphoreType.DMA for _ in range(NBUF)],
            [pltpu.SemaphoreType.DMA for _ in range(NBUF)],
        ],
    )
    def k(idx_hbm, table_hbm, out_hbm, idx_v, rows, gsems, osems):
        wid = lax.axis_index("s") * info.num_cores + lax.axis_index("c")
        base = wid * seq_per_w
        pltpu.sync_copy(idx_hbm.at[pl.ds(base, seq_per_w)], idx_v)

        def gather_copies(s, b):
            return (
                pltpu.make_async_copy(
                    table_hbm.at[idx_v.at[s, pl.ds(0, SPLIT)]],
                    rows[b].at[pl.ds(0, SPLIT)],
                    gsems[b],
                ),
                pltpu.make_async_copy(
                    table_hbm.at[idx_v.at[s, pl.ds(SPLIT, rest)]],
                    rows[b].at[pl.ds(SPLIT, rest)],
                    gsems[b],
                ),
            )

        def gather_start(s, b):
            c1, c2 = gather_copies(s, b)
            c1.start()
            c2.start()

        def gather_wait(s, b):
            c1, c2 = gather_copies(s, b)
            c1.wait()
            c2.wait()

        def out_copy(s, b):
            return pltpu.make_async_copy(rows[b], out_hbm.at[base + s], osems[b])

        def scale_buf(b):
            buf = rows[b]

            @plsc.parallel_loop(0, seq, unroll=4)
            def _(i):
                for j in range(D // 16):
                    sl = pl.ds(j * 16, 16)
                    buf[i, sl] = buf[i, sl] * SCALE

        for b in range(NBUF):
            gather_start(b, b)

        def body(q, carry):
            s0 = q * NBUF
            for b in range(NBUF):
                gather_wait(s0 + b, b)
                scale_buf(b)
                out_copy(s0 + b, b).start()
            for b in range(NBUF):
                out_copy(s0 + b, b).wait()
                gather_start(s0 + NBUF + b, b)
            return carry

        lax.fori_loop(0, n_quads - 1, body, 0)

        s0 = (n_quads - 1) * NBUF
        for b in range(NBUF):
            gather_wait(s0 + b, b)
            scale_buf(b)
            out_copy(s0 + b, b).start()
        for b in range(NBUF):
            out_copy(s0 + b, b).wait()

    return k


def kernel(xb, table):
    r, c = xb.shape
    return _make_sc_kernel(r, c)(xb.astype(jnp.int32), table)
